# Initial kernel scaffold; baseline (speedup 1.0000x reference)
import jax
import jax.numpy as jnp
from jax.experimental import pallas as pl


def kernel(mask_true, mask_pred):
    raise NotImplementedError("write your pallas kernel here")


# speedup vs baseline: 1.8601x; 1.8601x over previous
import functools

import jax
import jax.numpy as jnp
from jax.experimental import pallas as pl
from jax.experimental.pallas import tpu as pltpu

EPS = 1e-7
LABEL_SMOOTHING = 0.1


def _body(mt_ref, mp_ref, out_ref, acc_ref):
    b = pl.program_id(0)
    n = pl.program_id(1)

    @pl.when(n == 0)
    def _init():
        acc_ref[0] = 0.0
        acc_ref[1] = 0.0

    mt = mt_ref[0, 0]
    c = jnp.min(mt)
    valid = c < 80
    sc = jnp.where(valid, c, 0)

    pred = mp_ref[0, 0]
    lane = jax.lax.broadcasted_iota(jnp.int32, pred.shape, 2)
    chosen_pred = jnp.sum(jnp.where(lane == sc, pred, 0.0), axis=-1)
    chosen_true = (mt == sc).astype(jnp.float32)

    y = (1.0 - LABEL_SMOOTHING) * chosen_true + LABEL_SMOOTHING / 2.0
    loss = -(y * jnp.log(chosen_pred + EPS)
             + (1.0 - y) * jnp.log(1.0 - chosen_pred + EPS))
    molded = jnp.where(valid, jnp.mean(loss), 0.0)

    acc_ref[0] += molded
    acc_ref[1] += jnp.where(molded != 0.0, 1.0, 0.0)

    @pl.when(n == pl.num_programs(1) - 1)
    def _fin():
        out_ref[b] = acc_ref[0] / (acc_ref[1] + 1.0)


@jax.jit
def kernel(mask_true, mask_pred):
    B, N, H, W = mask_true.shape
    out = pl.pallas_call(
        _body,
        grid=(B, N),
        in_specs=[
            pl.BlockSpec((1, 1, H, W), lambda b, n: (b, n, 0, 0)),
            pl.BlockSpec((1, 1, H, W, mask_pred.shape[-1]),
                         lambda b, n: (b, n, 0, 0, 0)),
        ],
        out_specs=pl.BlockSpec(memory_space=pltpu.SMEM),
        out_shape=jax.ShapeDtypeStruct((B,), jnp.float32),
        scratch_shapes=[pltpu.SMEM((2,), jnp.float32)],
    )(mask_true, mask_pred)
    return out
